# Initial kernel scaffold; baseline (speedup 1.0000x reference)
import jax
import jax.numpy as jnp
from jax.experimental import pallas as pl


def kernel(src_points, ref_points, src_points_c, ref_points_c, src_node_corr_indices, ref_node_corr_indices, gt_node_corr_indices, gt_node_corr_overlaps, src_back_indices, ref_back_indices):
    raise NotImplementedError("write your pallas kernel here")


# speedup vs baseline: 8.5798x; 8.5798x over previous
import math

import jax
import jax.numpy as jnp
from jax.experimental import pallas as pl
from jax.experimental.pallas import tpu as pltpu

_Q = 1024
_B = 1024
_BB = 2048


def _nn_body(q2_ref, qq_ref, pt_ref, pp_ref, out_ref, rm_ref, ri_ref):
    j = pl.program_id(0)
    nb = pl.num_programs(0)

    @pl.when(j == 0)
    def _init():
        rm_ref[...] = jnp.full(rm_ref.shape, jnp.inf, jnp.float32)
        ri_ref[...] = jnp.zeros(ri_ref.shape, jnp.int32)

    mm2 = jax.lax.dot_general(
        q2_ref[...], pt_ref[...], (((1,), (0,)), ((), ())),
        preferred_element_type=jnp.float32)
    t = qq_ref[...] + pp_ref[...]
    d2 = t - mm2
    cols = j * _B + jax.lax.broadcasted_iota(jnp.int32, (_Q, _B), 1)
    rm = rm_ref[...]
    lt = d2 < rm
    rm_ref[...] = jnp.where(lt, d2, rm)
    ri_ref[...] = jnp.where(lt, cols, ri_ref[...])

    @pl.when(j == nb - 1)
    def _fin():
        rmf = rm_ref[...]
        best = jnp.min(rmf, axis=1, keepdims=True)
        ii = jnp.where(rmf == best, ri_ref[...], jnp.int32(2**30))
        out_ref[...] = jnp.min(ii, axis=1, keepdims=True)


def _nn_argmin(queries, points):
    n = points.shape[0]
    nb = -(-n // _B)
    npad = nb * _B
    q2 = (queries * 2.0)
    q2 = jnp.pad(q2, ((0, 0), (0, 5)))
    qq = jnp.sum(queries * queries, axis=1, keepdims=True)
    pt = jnp.pad(points.T, ((0, 5), (0, npad - n)))
    pp = jnp.sum(points * points, axis=1)
    pp = jnp.pad(pp, (0, npad - n), constant_values=jnp.inf)[None, :]
    out = pl.pallas_call(
        _nn_body,
        grid=(nb,),
        in_specs=[
            pl.BlockSpec((_Q, 8), lambda j: (0, 0)),
            pl.BlockSpec((_Q, 1), lambda j: (0, 0)),
            pl.BlockSpec((8, _B), lambda j: (0, j)),
            pl.BlockSpec((1, _B), lambda j: (0, j)),
        ],
        out_specs=pl.BlockSpec((_Q, 1), lambda j: (0, 0)),
        out_shape=jax.ShapeDtypeStruct((_Q, 1), jnp.int32),
        scratch_shapes=[
            pltpu.VMEM((_Q, _B), jnp.float32),
            pltpu.VMEM((_Q, _B), jnp.int32),
        ],
    )(q2, qq, pt, pp)
    return out


def _isin_body(idx_ref, back_ref, out_ref):
    j = pl.program_id(0)

    @pl.when(j == 0)
    def _init():
        out_ref[...] = jnp.zeros(out_ref.shape, jnp.float32)

    eq = idx_ref[...] == back_ref[...]
    hit = jnp.max(jnp.where(eq, 1.0, 0.0), axis=1, keepdims=True)
    out_ref[...] = jnp.maximum(out_ref[...], hit)


def _isin(idx_col, back):
    nb = -(-back.shape[0] // _BB)
    npad = nb * _BB
    backp = jnp.pad(back, (0, npad - back.shape[0]), constant_values=-1)[None, :]
    return pl.pallas_call(
        _isin_body,
        grid=(nb,),
        in_specs=[
            pl.BlockSpec((_Q, 1), lambda j: (0, 0)),
            pl.BlockSpec((1, _BB), lambda j: (0, j)),
        ],
        out_specs=pl.BlockSpec((_Q, 1), lambda j: (0, 0)),
        out_shape=jax.ShapeDtypeStruct((_Q, 1), jnp.float32),
    )(idx_col, backp)


def _pairs_body(mr_r, mr_c, ms_r, ms_c, refn_c, refn_r, srcn_c, srcn_r,
                gx_c, gx_r, gy_c, gy_r, ov_c, ov_r,
                loss_ref, loss1_ref, loss2_ref, gtm_ref):
    f32 = jnp.float32
    keC = refn_c[...] * 1024 + srcn_c[...]
    keR = refn_r[...] * 1024 + srcn_r[...]
    eqE = keC == keR
    iE_r = jax.lax.broadcasted_iota(jnp.int32, (512, 512), 0)
    iE_c = jax.lax.broadcasted_iota(jnp.int32, (512, 512), 1)
    dupE = jnp.any(eqE & (iE_c < iE_r), axis=1, keepdims=True)
    n_pos = jnp.sum(jnp.where(dupE, f32(0), f32(1)))

    lane_g = jax.lax.broadcasted_iota(jnp.int32, (2048, 1024), 1)
    mrg = jnp.sum(jnp.where(lane_g == gx_c[...], mr_r[...], f32(0)),
                  axis=1, keepdims=True)
    msg = jnp.sum(jnp.where(lane_g == gy_c[...], ms_r[...], f32(0)),
                  axis=1, keepdims=True)
    mC = (mrg * msg) > 0
    sub_g = jax.lax.broadcasted_iota(jnp.int32, (1024, 2048), 0)
    mrgR = jnp.sum(jnp.where(sub_g == gx_r[...], mr_c[...], f32(0)),
                   axis=0, keepdims=True)
    msgR = jnp.sum(jnp.where(sub_g == gy_r[...], ms_c[...], f32(0)),
                   axis=0, keepdims=True)
    mR = (mrgR * msgR) > 0

    kgC = gx_c[...] * 1024 + gy_c[...]
    kgR = gx_r[...] * 1024 + gy_r[...]
    eqG = kgC == kgR
    iG_r = jax.lax.broadcasted_iota(jnp.int32, (2048, 2048), 0)
    iG_c = jax.lax.broadcasted_iota(jnp.int32, (2048, 2048), 1)
    lower = iG_c < iG_r
    upper = iG_c > iG_r
    firstC = ~jnp.any(eqG & lower, axis=1, keepdims=True)
    lastC = ~jnp.any(eqG & upper, axis=1, keepdims=True)
    firstR = ~jnp.any(eqG & upper, axis=0, keepdims=True)
    lastR = ~jnp.any(eqG & lower, axis=0, keepdims=True)
    activeC = firstC & mC
    activeR = firstR & mR
    n_active = jnp.sum(jnp.where(activeC, f32(1), f32(0)))
    vC = jnp.sum(jnp.where(eqG & lastR, ov_r[...], f32(0)),
                 axis=1, keepdims=True)
    vR = jnp.sum(jnp.where(eqG & lastC, ov_c[...], f32(0)),
                 axis=0, keepdims=True)

    beats = activeR & ((vR > vC) | ((vR == vC) & (kgR < kgC)))
    rank = jnp.sum(jnp.where(beats, f32(1), f32(0)), axis=1, keepdims=True)
    selF = jnp.where(activeC & (rank < 256.0), f32(1), f32(0))
    actF = jnp.where(activeC, f32(1), f32(0))
    use_topk = n_active > 256.0
    gsel = jnp.where(use_topk, selF, actF)
    n_gt = jnp.sum(gsel)
    eshit = jnp.any(kgC == keR, axis=1, keepdims=True)
    n_both = jnp.sum(jnp.where(eshit, gsel, f32(0)))

    nm = f32(1024 * 1024)
    n_pos_c = jnp.maximum(n_pos, f32(1))
    ratio = 1.0 / (n_pos_c / nm)
    sum_abs = n_gt + n_pos - 2.0 * n_both
    loss1 = f32(math.sqrt(2)) * ratio * (sum_abs / nm)
    loss1_ref[0, 0] = loss1
    loss2_ref[0, 0] = f32(0)
    loss_ref[0, 0] = loss1 + f32(0)
    gtm_ref[...] = 1.0 - jnp.concatenate([mr_r[...], ms_r[...]], axis=1)


def _pairs(mask_ref_v, mask_src_v, refn, srcn, gx, gy, ov):
    mr_r = mask_ref_v.reshape(1, 1024)
    mr_c = mask_ref_v.reshape(1024, 1)
    ms_r = mask_src_v.reshape(1, 1024)
    ms_c = mask_src_v.reshape(1024, 1)
    args = (mr_r, mr_c, ms_r, ms_c,
            refn.reshape(512, 1), refn.reshape(1, 512),
            srcn.reshape(512, 1), srcn.reshape(1, 512),
            gx.reshape(2048, 1), gx.reshape(1, 2048),
            gy.reshape(2048, 1), gy.reshape(1, 2048),
            ov.reshape(2048, 1), ov.reshape(1, 2048))
    specs = [pl.BlockSpec(a.shape, lambda: (0, 0)) for a in args]
    outs = pl.pallas_call(
        _pairs_body,
        in_specs=specs,
        out_specs=[
            pl.BlockSpec(memory_space=pltpu.SMEM),
            pl.BlockSpec(memory_space=pltpu.SMEM),
            pl.BlockSpec(memory_space=pltpu.SMEM),
            pl.BlockSpec((1, 2048), lambda: (0, 0)),
        ],
        out_shape=[
            jax.ShapeDtypeStruct((1, 1), jnp.float32),
            jax.ShapeDtypeStruct((1, 1), jnp.float32),
            jax.ShapeDtypeStruct((1, 1), jnp.float32),
            jax.ShapeDtypeStruct((1, 2048), jnp.float32),
        ],
    )(*args)
    return outs


def kernel(src_points, ref_points, src_points_c, ref_points_c,
           src_node_corr_indices, ref_node_corr_indices,
           gt_node_corr_indices, gt_node_corr_overlaps,
           src_back_indices, ref_back_indices):
    idx_src = _nn_argmin(src_points_c, src_points)
    idx_ref = _nn_argmin(ref_points_c, ref_points)
    mask_src = _isin(idx_src, src_back_indices)
    mask_ref = _isin(idx_ref, ref_back_indices)
    loss, loss1, loss2, inv_gtm = _pairs(
        mask_ref.reshape(-1), mask_src.reshape(-1),
        ref_node_corr_indices, src_node_corr_indices,
        gt_node_corr_indices[:, 0], gt_node_corr_indices[:, 1],
        gt_node_corr_overlaps)
    return (loss.reshape(()), loss1.reshape(()), loss2.reshape(()),
            inv_gtm.reshape(2048))


# speedup vs baseline: 8.7836x; 1.0238x over previous
import math

import jax
import jax.numpy as jnp
from jax.experimental import pallas as pl
from jax.experimental.pallas import tpu as pltpu

_Q = 1024
_B = 2048
_BB = 2048
_NS = 30
_NR = 25
_MS = 15
_MR = 13


def _nn_body(q2_ref, qq_ref, pt_ref, pp_ref, outs_ref, outr_ref, rm_ref, ri_ref):
    j = pl.program_id(0)

    @pl.when((j == 0) | (j == _NS))
    def _init():
        rm_ref[...] = jnp.full(rm_ref.shape, jnp.inf, jnp.float32)
        ri_ref[...] = jnp.zeros(ri_ref.shape, jnp.int32)

    mm2 = jax.lax.dot_general(
        q2_ref[0], pt_ref[...], (((1,), (0,)), ((), ())),
        preferred_element_type=jnp.float32)
    t = qq_ref[0] + pp_ref[...]
    d2 = t - mm2
    rm = rm_ref[...]
    lt = d2 < rm
    rm_ref[...] = jnp.where(lt, d2, rm)
    ri_ref[...] = jnp.where(lt, j, ri_ref[...])

    def _argmin(base):
        rmf = rm_ref[...]
        pos = jax.lax.broadcasted_iota(jnp.int32, (_Q, _B), 1)
        gidx = (ri_ref[...] - base) * _B + pos
        best = jnp.min(rmf, axis=1, keepdims=True)
        ii = jnp.where(rmf == best, gidx, jnp.int32(2**30))
        return jnp.min(ii, axis=1, keepdims=True)

    @pl.when(j == _NS - 1)
    def _fin_src():
        outs_ref[...] = _argmin(0)

    @pl.when(j == _NS + _NR - 1)
    def _fin_ref():
        outr_ref[...] = _argmin(_NS)


def _nn_argmin(src_c, src_pts, ref_c, ref_pts):
    def prep(queries, points, npad):
        n = points.shape[0]
        q2 = jnp.pad(queries * 2.0, ((0, 0), (0, 5)))
        qq = jnp.sum(queries * queries, axis=1, keepdims=True)
        pt = jnp.pad(points.T, ((0, 5), (0, npad - n)))
        pp = jnp.sum(points * points, axis=1)
        pp = jnp.pad(pp, (0, npad - n), constant_values=jnp.inf)[None, :]
        return q2, qq, pt, pp

    q2s, qqs, pts, pps = prep(src_c, src_pts, _NS * _B)
    q2r, qqr, ptr, ppr = prep(ref_c, ref_pts, _NR * _B)
    q2 = jnp.stack([q2s, q2r])
    qq = jnp.stack([qqs, qqr])
    pt = jnp.concatenate([pts, ptr], axis=1)
    pp = jnp.concatenate([pps, ppr], axis=1)

    cloud = lambda j: jnp.where(j < _NS, 0, 1)
    return pl.pallas_call(
        _nn_body,
        grid=(_NS + _NR,),
        in_specs=[
            pl.BlockSpec((1, _Q, 8), lambda j: (cloud(j), 0, 0)),
            pl.BlockSpec((1, _Q, 1), lambda j: (cloud(j), 0, 0)),
            pl.BlockSpec((8, _B), lambda j: (0, j)),
            pl.BlockSpec((1, _B), lambda j: (0, j)),
        ],
        out_specs=[
            pl.BlockSpec((_Q, 1), lambda j: (0, 0)),
            pl.BlockSpec((_Q, 1), lambda j: (0, 0)),
        ],
        out_shape=[
            jax.ShapeDtypeStruct((_Q, 1), jnp.int32),
            jax.ShapeDtypeStruct((_Q, 1), jnp.int32),
        ],
        scratch_shapes=[
            pltpu.VMEM((_Q, _B), jnp.float32),
            pltpu.VMEM((_Q, _B), jnp.int32),
        ],
    )(q2, qq, pt, pp)


def _isin_body(idx_ref, back_ref, outs_ref, outr_ref):
    j = pl.program_id(0)

    @pl.when(j == 0)
    def _init_s():
        outs_ref[...] = jnp.zeros(outs_ref.shape, jnp.float32)

    @pl.when(j == _MS)
    def _init_r():
        outr_ref[...] = jnp.zeros(outr_ref.shape, jnp.float32)

    eq = idx_ref[0] == back_ref[...]
    hit = jnp.max(jnp.where(eq, 1.0, 0.0), axis=1, keepdims=True)

    @pl.when(j < _MS)
    def _acc_s():
        outs_ref[...] = jnp.maximum(outs_ref[...], hit)

    @pl.when(j >= _MS)
    def _acc_r():
        outr_ref[...] = jnp.maximum(outr_ref[...], hit)


def _isin(idx_src, src_back, idx_ref, ref_back):
    backs = jnp.pad(src_back, (0, _MS * _BB - src_back.shape[0]),
                    constant_values=-1)
    backr = jnp.pad(ref_back, (0, _MR * _BB - ref_back.shape[0]),
                    constant_values=-1)
    back = jnp.concatenate([backs, backr])[None, :]
    idx = jnp.stack([idx_src, idx_ref])
    cloud = lambda j: jnp.where(j < _MS, 0, 1)
    return pl.pallas_call(
        _isin_body,
        grid=(_MS + _MR,),
        in_specs=[
            pl.BlockSpec((1, _Q, 1), lambda j: (cloud(j), 0, 0)),
            pl.BlockSpec((1, _BB), lambda j: (0, j)),
        ],
        out_specs=[
            pl.BlockSpec((_Q, 1), lambda j: (0, 0)),
            pl.BlockSpec((_Q, 1), lambda j: (0, 0)),
        ],
        out_shape=[
            jax.ShapeDtypeStruct((_Q, 1), jnp.float32),
            jax.ShapeDtypeStruct((_Q, 1), jnp.float32),
        ],
    )(idx, back)


def _pairs_body(mr_r, mr_c, ms_r, ms_c, refn_c, refn_r, srcn_c, srcn_r,
                gx_c, gx_r, gy_c, gy_r, ov_c, ov_r,
                loss_ref, loss1_ref, loss2_ref, gtm_ref):
    f32 = jnp.float32
    keC = refn_c[...] * 1024 + srcn_c[...]
    keR = refn_r[...] * 1024 + srcn_r[...]
    eqE = keC == keR
    iE_r = jax.lax.broadcasted_iota(jnp.int32, (512, 512), 0)
    iE_c = jax.lax.broadcasted_iota(jnp.int32, (512, 512), 1)
    dupE = jnp.any(eqE & (iE_c < iE_r), axis=1, keepdims=True)
    n_pos = jnp.sum(jnp.where(dupE, f32(0), f32(1)))

    lane_g = jax.lax.broadcasted_iota(jnp.int32, (2048, 1024), 1)
    mrg = jnp.sum(jnp.where(lane_g == gx_c[...], mr_r[...], f32(0)),
                  axis=1, keepdims=True)
    msg = jnp.sum(jnp.where(lane_g == gy_c[...], ms_r[...], f32(0)),
                  axis=1, keepdims=True)
    mC = (mrg * msg) > 0
    sub_g = jax.lax.broadcasted_iota(jnp.int32, (1024, 2048), 0)
    mrgR = jnp.sum(jnp.where(sub_g == gx_r[...], mr_c[...], f32(0)),
                   axis=0, keepdims=True)
    msgR = jnp.sum(jnp.where(sub_g == gy_r[...], ms_c[...], f32(0)),
                   axis=0, keepdims=True)
    mR = (mrgR * msgR) > 0

    kgC = gx_c[...] * 1024 + gy_c[...]
    kgR = gx_r[...] * 1024 + gy_r[...]
    eqG = kgC == kgR
    iG_r = jax.lax.broadcasted_iota(jnp.int32, (2048, 2048), 0)
    iG_c = jax.lax.broadcasted_iota(jnp.int32, (2048, 2048), 1)
    lower = eqG & (iG_c < iG_r)
    upper = eqG & (iG_c > iG_r)
    firstC = ~jnp.any(lower, axis=1, keepdims=True)
    lastC = ~jnp.any(upper, axis=1, keepdims=True)
    firstR = ~jnp.any(eqG & (iG_r < iG_c), axis=0, keepdims=True)
    lastR = ~jnp.any(eqG & (iG_r > iG_c), axis=0, keepdims=True)
    activeC = firstC & mC
    activeR = firstR & mR
    actF = jnp.where(activeC, f32(1), f32(0))
    n_active = jnp.sum(actF)
    vC = jnp.sum(jnp.where(eqG & lastR, ov_r[...], f32(0)),
                 axis=1, keepdims=True)
    vR = jnp.sum(jnp.where(eqG & lastC, ov_c[...], f32(0)),
                 axis=0, keepdims=True)

    beats = activeR & ((vR > vC) | ((vR == vC) & (kgR < kgC)))
    rank = jnp.sum(jnp.where(beats, f32(1), f32(0)), axis=1, keepdims=True)
    selF = jnp.where(activeC & (rank < 256.0), f32(1), f32(0))
    use_topk = n_active > 256.0
    gsel = jnp.where(use_topk, selF, actF)
    n_gt = jnp.sum(gsel)
    eshit = jnp.any(kgC == keR, axis=1, keepdims=True)
    n_both = jnp.sum(jnp.where(eshit, gsel, f32(0)))

    nm = f32(1024 * 1024)
    n_pos_c = jnp.maximum(n_pos, f32(1))
    ratio = 1.0 / (n_pos_c / nm)
    sum_abs = n_gt + n_pos - 2.0 * n_both
    loss1 = f32(math.sqrt(2)) * ratio * (sum_abs / nm)
    loss1_ref[0, 0] = loss1
    loss2_ref[0, 0] = f32(0)
    loss_ref[0, 0] = loss1 + f32(0)
    gtm_ref[...] = 1.0 - jnp.concatenate([mr_r[...], ms_r[...]], axis=1)


def _pairs(mask_ref_v, mask_src_v, refn, srcn, gx, gy, ov):
    args = (mask_ref_v.reshape(1, 1024), mask_ref_v.reshape(1024, 1),
            mask_src_v.reshape(1, 1024), mask_src_v.reshape(1024, 1),
            refn.reshape(512, 1), refn.reshape(1, 512),
            srcn.reshape(512, 1), srcn.reshape(1, 512),
            gx.reshape(2048, 1), gx.reshape(1, 2048),
            gy.reshape(2048, 1), gy.reshape(1, 2048),
            ov.reshape(2048, 1), ov.reshape(1, 2048))
    specs = [pl.BlockSpec(a.shape, lambda: (0, 0)) for a in args]
    return pl.pallas_call(
        _pairs_body,
        in_specs=specs,
        out_specs=[
            pl.BlockSpec(memory_space=pltpu.SMEM),
            pl.BlockSpec(memory_space=pltpu.SMEM),
            pl.BlockSpec(memory_space=pltpu.SMEM),
            pl.BlockSpec((1, 2048), lambda: (0, 0)),
        ],
        out_shape=[
            jax.ShapeDtypeStruct((1, 1), jnp.float32),
            jax.ShapeDtypeStruct((1, 1), jnp.float32),
            jax.ShapeDtypeStruct((1, 1), jnp.float32),
            jax.ShapeDtypeStruct((1, 2048), jnp.float32),
        ],
    )(*args)


def kernel(src_points, ref_points, src_points_c, ref_points_c,
           src_node_corr_indices, ref_node_corr_indices,
           gt_node_corr_indices, gt_node_corr_overlaps,
           src_back_indices, ref_back_indices):
    idx_src, idx_ref = _nn_argmin(src_points_c, src_points,
                                  ref_points_c, ref_points)
    mask_src, mask_ref = _isin(idx_src, src_back_indices,
                               idx_ref, ref_back_indices)
    loss, loss1, loss2, inv_gtm = _pairs(
        mask_ref.reshape(-1), mask_src.reshape(-1),
        ref_node_corr_indices, src_node_corr_indices,
        gt_node_corr_indices[:, 0], gt_node_corr_indices[:, 1],
        gt_node_corr_overlaps)
    return (loss.reshape(()), loss1.reshape(()), loss2.reshape(()),
            inv_gtm.reshape(2048))


# speedup vs baseline: 9.3540x; 1.0649x over previous
import functools
import math

import jax
import jax.numpy as jnp
from jax.experimental import pallas as pl
from jax.experimental.pallas import tpu as pltpu
from jax.experimental.pallas import tpu_sc as plsc

_Q = 1024
_B = 2048
_BB = 2048
_NS = 30
_NR = 25
_MS = 15
_MR = 13


def _nn_body(q2_ref, qq_ref, pt_ref, pp_ref, outs_ref, outr_ref, rm_ref, ri_ref):
    j = pl.program_id(0)

    @pl.when((j == 0) | (j == _NS))
    def _init():
        rm_ref[...] = jnp.full(rm_ref.shape, jnp.inf, jnp.float32)
        ri_ref[...] = jnp.zeros(ri_ref.shape, jnp.int32)

    mm2 = jax.lax.dot_general(
        q2_ref[0], pt_ref[...], (((1,), (0,)), ((), ())),
        preferred_element_type=jnp.float32)
    t = qq_ref[0] + pp_ref[...]
    d2 = t - mm2
    rm = rm_ref[...]
    lt = d2 < rm
    rm_ref[...] = jnp.minimum(d2, rm)
    ri_ref[...] = jnp.where(lt, j, ri_ref[...])

    def _argmin(base):
        rmf = rm_ref[...]
        pos = jax.lax.broadcasted_iota(jnp.int32, (_Q, _B), 1)
        gidx = (ri_ref[...] - base) * _B + pos
        best = jnp.min(rmf, axis=1, keepdims=True)
        ii = jnp.where(rmf == best, gidx, jnp.int32(2**30))
        return jnp.min(ii, axis=1, keepdims=True)

    @pl.when(j == _NS - 1)
    def _fin_src():
        outs_ref[...] = _argmin(0)

    @pl.when(j == _NS + _NR - 1)
    def _fin_ref():
        outr_ref[...] = _argmin(_NS)


def _nn_argmin(src_c, src_pts, ref_c, ref_pts):
    def prep(queries, points, npad):
        n = points.shape[0]
        q2 = jnp.pad(queries * 2.0, ((0, 0), (0, 5)))
        qq = jnp.sum(queries * queries, axis=1, keepdims=True)
        pt = jnp.pad(points.T, ((0, 5), (0, npad - n)))
        pp = jnp.sum(points * points, axis=1)
        pp = jnp.pad(pp, (0, npad - n), constant_values=jnp.inf)[None, :]
        return q2, qq, pt, pp

    q2s, qqs, pts, pps = prep(src_c, src_pts, _NS * _B)
    q2r, qqr, ptr, ppr = prep(ref_c, ref_pts, _NR * _B)
    q2 = jnp.stack([q2s, q2r])
    qq = jnp.stack([qqs, qqr])
    pt = jnp.concatenate([pts, ptr], axis=1)
    pp = jnp.concatenate([pps, ppr], axis=1)

    cloud = lambda j: jnp.where(j < _NS, 0, 1)
    return pl.pallas_call(
        _nn_body,
        grid=(_NS + _NR,),
        in_specs=[
            pl.BlockSpec((1, _Q, 8), lambda j: (cloud(j), 0, 0)),
            pl.BlockSpec((1, _Q, 1), lambda j: (cloud(j), 0, 0)),
            pl.BlockSpec((8, _B), lambda j: (0, j)),
            pl.BlockSpec((1, _B), lambda j: (0, j)),
        ],
        out_specs=[
            pl.BlockSpec((_Q, 1), lambda j: (0, 0)),
            pl.BlockSpec((_Q, 1), lambda j: (0, 0)),
        ],
        out_shape=[
            jax.ShapeDtypeStruct((_Q, 1), jnp.int32),
            jax.ShapeDtypeStruct((_Q, 1), jnp.int32),
        ],
        scratch_shapes=[
            pltpu.VMEM((_Q, _B), jnp.float32),
            pltpu.VMEM((_Q, _B), jnp.int32),
        ],
    )(q2, qq, pt, pp)


_SC_MESH = dict(core_axis_name="c", subcore_axis_name="s",
                num_cores=2, num_subcores=16)
_BMS = 61440
_BMR = 51200


def _sc_scatter_body(sback, rback, zeros_h, ones_h, bms, bmr,
                     shared, idx_s, ones_s, idx_r, ones_r):
    c = jax.lax.axis_index("c")
    s = jax.lax.axis_index("s")

    @pl.when(c == 0)
    def _src():
        zl = _BMS // 16
        pltpu.sync_copy(zeros_h.at[pl.ds(s * zl, zl)],
                        shared.at[pl.ds(s * zl, zl)])
        pltpu.sync_copy(sback.at[pl.ds(s * 1920, 1920)], idx_s)
        pltpu.sync_copy(ones_h.at[pl.ds(0, 1920)], ones_s)
        plsc.subcore_barrier()
        pltpu.sync_copy(ones_s, shared.at[idx_s])
        plsc.subcore_barrier()
        pltpu.sync_copy(shared.at[pl.ds(s * zl, zl)],
                        bms.at[pl.ds(s * zl, zl)])

    @pl.when(c == 1)
    def _ref():
        zl = _BMR // 16
        pltpu.sync_copy(zeros_h.at[pl.ds(s * zl, zl)],
                        shared.at[pl.ds(s * zl, zl)])
        pltpu.sync_copy(rback.at[pl.ds(s * 1600, 1600)], idx_r)
        pltpu.sync_copy(ones_h.at[pl.ds(0, 1600)], ones_r)
        plsc.subcore_barrier()
        pltpu.sync_copy(ones_r, shared.at[idx_r])
        plsc.subcore_barrier()
        pltpu.sync_copy(shared.at[pl.ds(s * zl, zl)],
                        bmr.at[pl.ds(s * zl, zl)])


def _sc_scatter(src_back, ref_back):
    sback = jnp.pad(src_back, (0, 30720 - src_back.shape[0]),
                    constant_values=_BMS - 1)
    rback = jnp.pad(ref_back, (0, 25600 - ref_back.shape[0]),
                    constant_values=_BMR - 1)
    zeros_h = jnp.zeros((_BMS,), jnp.float32)
    ones_h = jnp.ones((1920,), jnp.float32)
    f = pl.kernel(
        _sc_scatter_body,
        out_type=[
            jax.ShapeDtypeStruct((_BMS,), jnp.float32),
            jax.ShapeDtypeStruct((_BMR,), jnp.float32),
        ],
        mesh=plsc.VectorSubcoreMesh(**_SC_MESH),
        scratch_types=[
            pltpu.VMEM_SHARED((_BMS,), jnp.float32),
            pltpu.VMEM((1920,), jnp.int32),
            pltpu.VMEM((1920,), jnp.float32),
            pltpu.VMEM((1600,), jnp.int32),
            pltpu.VMEM((1600,), jnp.float32),
        ],
    )
    return f(sback, rback, zeros_h, ones_h)


def _sc_gather_body(bms, bmr, idxs, idxr, masks, maskr, qi, qv, sem):
    c = jax.lax.axis_index("c")
    s = jax.lax.axis_index("s")
    w = s * 2 + c
    pltpu.sync_copy(idxs.at[pl.ds(w * 32, 32)], qi)
    pltpu.async_copy(bms.at[qi], qv, sem).wait()
    pltpu.sync_copy(qv, masks.at[pl.ds(w * 32, 32)])
    pltpu.sync_copy(idxr.at[pl.ds(w * 32, 32)], qi)
    pltpu.async_copy(bmr.at[qi], qv, sem).wait()
    pltpu.sync_copy(qv, maskr.at[pl.ds(w * 32, 32)])


def _sc_gather(bm_src, bm_ref, idx_src, idx_ref):
    f = pl.kernel(
        _sc_gather_body,
        out_type=[
            jax.ShapeDtypeStruct((_Q,), jnp.float32),
            jax.ShapeDtypeStruct((_Q,), jnp.float32),
        ],
        mesh=plsc.VectorSubcoreMesh(**_SC_MESH),
        scratch_types=[
            pltpu.VMEM((32,), jnp.int32),
            pltpu.VMEM((32,), jnp.float32),
            pltpu.SemaphoreType.DMA,
        ],
    )
    return f(bm_src, bm_ref, idx_src, idx_ref)


def _isin_body(idx_ref, back_ref, outs_ref, outr_ref):
    j = pl.program_id(0)

    @pl.when(j == 0)
    def _init_s():
        outs_ref[...] = jnp.zeros(outs_ref.shape, jnp.float32)

    @pl.when(j == _MS)
    def _init_r():
        outr_ref[...] = jnp.zeros(outr_ref.shape, jnp.float32)

    eq = idx_ref[0] == back_ref[...]
    hit = jnp.max(jnp.where(eq, 1.0, 0.0), axis=1, keepdims=True)

    @pl.when(j < _MS)
    def _acc_s():
        outs_ref[...] = jnp.maximum(outs_ref[...], hit)

    @pl.when(j >= _MS)
    def _acc_r():
        outr_ref[...] = jnp.maximum(outr_ref[...], hit)


def _isin(idx_src, src_back, idx_ref, ref_back):
    backs = jnp.pad(src_back, (0, _MS * _BB - src_back.shape[0]),
                    constant_values=-1)
    backr = jnp.pad(ref_back, (0, _MR * _BB - ref_back.shape[0]),
                    constant_values=-1)
    back = jnp.concatenate([backs, backr])[None, :]
    idx = jnp.stack([idx_src, idx_ref])
    cloud = lambda j: jnp.where(j < _MS, 0, 1)
    return pl.pallas_call(
        _isin_body,
        grid=(_MS + _MR,),
        in_specs=[
            pl.BlockSpec((1, _Q, 1), lambda j: (cloud(j), 0, 0)),
            pl.BlockSpec((1, _BB), lambda j: (0, j)),
        ],
        out_specs=[
            pl.BlockSpec((_Q, 1), lambda j: (0, 0)),
            pl.BlockSpec((_Q, 1), lambda j: (0, 0)),
        ],
        out_shape=[
            jax.ShapeDtypeStruct((_Q, 1), jnp.float32),
            jax.ShapeDtypeStruct((_Q, 1), jnp.float32),
        ],
    )(idx, back)


def _pairs_body(mr_r, mr_c, ms_r, ms_c, refn_c, refn_r, srcn_c, srcn_r,
                gx_c, gx_r, gy_c, gy_r, ov_c, ov_r,
                loss_ref, loss1_ref, loss2_ref, gtm_ref):
    f32 = jnp.float32
    keC = refn_c[...] * 1024 + srcn_c[...]
    keR = refn_r[...] * 1024 + srcn_r[...]
    eqE = keC == keR
    iE_r = jax.lax.broadcasted_iota(jnp.int32, (512, 512), 0)
    iE_c = jax.lax.broadcasted_iota(jnp.int32, (512, 512), 1)
    dupE = jnp.any(eqE & (iE_c < iE_r), axis=1, keepdims=True)
    n_pos = jnp.sum(jnp.where(dupE, f32(0), f32(1)))

    lane_g = jax.lax.broadcasted_iota(jnp.int32, (2048, 1024), 1)
    mrg = jnp.sum(jnp.where(lane_g == gx_c[...], mr_r[...], f32(0)),
                  axis=1, keepdims=True)
    msg = jnp.sum(jnp.where(lane_g == gy_c[...], ms_r[...], f32(0)),
                  axis=1, keepdims=True)
    mC = (mrg * msg) > 0
    sub_g = jax.lax.broadcasted_iota(jnp.int32, (1024, 2048), 0)
    mrgR = jnp.sum(jnp.where(sub_g == gx_r[...], mr_c[...], f32(0)),
                   axis=0, keepdims=True)
    msgR = jnp.sum(jnp.where(sub_g == gy_r[...], ms_c[...], f32(0)),
                   axis=0, keepdims=True)
    mR = (mrgR * msgR) > 0

    kgC = gx_c[...] * 1024 + gy_c[...]
    kgR = gx_r[...] * 1024 + gy_r[...]
    eqG = kgC == kgR
    iG_r = jax.lax.broadcasted_iota(jnp.int32, (2048, 2048), 0)
    iG_c = jax.lax.broadcasted_iota(jnp.int32, (2048, 2048), 1)
    lower = eqG & (iG_c < iG_r)
    upper = eqG & (iG_c > iG_r)
    firstC = ~jnp.any(lower, axis=1, keepdims=True)
    lastC = ~jnp.any(upper, axis=1, keepdims=True)
    firstR = ~jnp.any(eqG & (iG_r < iG_c), axis=0, keepdims=True)
    lastR = ~jnp.any(eqG & (iG_r > iG_c), axis=0, keepdims=True)
    activeC = firstC & mC
    activeR = firstR & mR
    actF = jnp.where(activeC, f32(1), f32(0))
    n_active = jnp.sum(actF)
    vC = jnp.sum(jnp.where(eqG & lastR, ov_r[...], f32(0)),
                 axis=1, keepdims=True)
    vR = jnp.sum(jnp.where(eqG & lastC, ov_c[...], f32(0)),
                 axis=0, keepdims=True)

    beats = activeR & ((vR > vC) | ((vR == vC) & (kgR < kgC)))
    rank = jnp.sum(jnp.where(beats, f32(1), f32(0)), axis=1, keepdims=True)
    selF = jnp.where(activeC & (rank < 256.0), f32(1), f32(0))
    use_topk = n_active > 256.0
    gsel = jnp.where(use_topk, selF, actF)
    n_gt = jnp.sum(gsel)
    eshit = jnp.any(kgC == keR, axis=1, keepdims=True)
    n_both = jnp.sum(jnp.where(eshit, gsel, f32(0)))

    nm = f32(1024 * 1024)
    n_pos_c = jnp.maximum(n_pos, f32(1))
    ratio = 1.0 / (n_pos_c / nm)
    sum_abs = n_gt + n_pos - 2.0 * n_both
    loss1 = f32(math.sqrt(2)) * ratio * (sum_abs / nm)
    loss1_ref[0, 0] = loss1
    loss2_ref[0, 0] = f32(0)
    loss_ref[0, 0] = loss1 + f32(0)
    gtm_ref[...] = 1.0 - jnp.concatenate([mr_r[...], ms_r[...]], axis=1)


def _pairs(mask_ref_v, mask_src_v, refn, srcn, gx, gy, ov):
    args = (mask_ref_v.reshape(1, 1024), mask_ref_v.reshape(1024, 1),
            mask_src_v.reshape(1, 1024), mask_src_v.reshape(1024, 1),
            refn.reshape(512, 1), refn.reshape(1, 512),
            srcn.reshape(512, 1), srcn.reshape(1, 512),
            gx.reshape(2048, 1), gx.reshape(1, 2048),
            gy.reshape(2048, 1), gy.reshape(1, 2048),
            ov.reshape(2048, 1), ov.reshape(1, 2048))
    specs = [pl.BlockSpec(a.shape, lambda: (0, 0)) for a in args]
    return pl.pallas_call(
        _pairs_body,
        in_specs=specs,
        out_specs=[
            pl.BlockSpec(memory_space=pltpu.SMEM),
            pl.BlockSpec(memory_space=pltpu.SMEM),
            pl.BlockSpec(memory_space=pltpu.SMEM),
            pl.BlockSpec((1, 2048), lambda: (0, 0)),
        ],
        out_shape=[
            jax.ShapeDtypeStruct((1, 1), jnp.float32),
            jax.ShapeDtypeStruct((1, 1), jnp.float32),
            jax.ShapeDtypeStruct((1, 1), jnp.float32),
            jax.ShapeDtypeStruct((1, 2048), jnp.float32),
        ],
    )(*args)


def kernel(src_points, ref_points, src_points_c, ref_points_c,
           src_node_corr_indices, ref_node_corr_indices,
           gt_node_corr_indices, gt_node_corr_overlaps,
           src_back_indices, ref_back_indices):
    bm_src, bm_ref = _sc_scatter(src_back_indices, ref_back_indices)
    idx_src, idx_ref = _nn_argmin(src_points_c, src_points,
                                  ref_points_c, ref_points)
    mask_src, mask_ref = _sc_gather(bm_src, bm_ref,
                                    idx_src.reshape(-1), idx_ref.reshape(-1))
    loss, loss1, loss2, inv_gtm = _pairs(
        mask_ref.reshape(-1), mask_src.reshape(-1),
        ref_node_corr_indices, src_node_corr_indices,
        gt_node_corr_indices[:, 0], gt_node_corr_indices[:, 1],
        gt_node_corr_overlaps)
    return (loss.reshape(()), loss1.reshape(()), loss2.reshape(()),
            inv_gtm.reshape(2048))


# speedup vs baseline: 9.4986x; 1.0155x over previous
import functools
import math

import jax
import jax.numpy as jnp
from jax.experimental import pallas as pl
from jax.experimental.pallas import tpu as pltpu
from jax.experimental.pallas import tpu_sc as plsc

_Q = 1024
_B = 2048
_BB = 2048
_NS = 30
_NR = 25
_MS = 15
_MR = 13


def _nn_body(q2_ref, qq_ref, pt_ref, pp_ref, outs_ref, outr_ref, rm_ref, ri_ref):
    j = pl.program_id(0)

    @pl.when((j == 0) | (j == _NS))
    def _init():
        rm_ref[...] = jnp.full(rm_ref.shape, jnp.inf, jnp.float32)
        ri_ref[...] = jnp.zeros(ri_ref.shape, jnp.int32)

    mm2 = jax.lax.dot_general(
        q2_ref[0], pt_ref[...], (((1,), (0,)), ((), ())),
        preferred_element_type=jnp.float32)
    t = qq_ref[0] + pp_ref[...]
    d2 = t - mm2
    rm = rm_ref[...]
    lt = d2 < rm
    rm_ref[...] = jnp.minimum(d2, rm)
    ri_ref[...] = jnp.where(lt, j, ri_ref[...])

    def _argmin(base):
        rmf = rm_ref[...]
        pos = jax.lax.broadcasted_iota(jnp.int32, (_Q, _B), 1)
        gidx = (ri_ref[...] - base) * _B + pos
        best = jnp.min(rmf, axis=1, keepdims=True)
        ii = jnp.where(rmf == best, gidx, jnp.int32(2**30))
        return jnp.min(ii, axis=1, keepdims=True)

    @pl.when(j == _NS - 1)
    def _fin_src():
        outs_ref[...] = _argmin(0)

    @pl.when(j == _NS + _NR - 1)
    def _fin_ref():
        outr_ref[...] = _argmin(_NS)


def _nn_argmin(src_c, src_pts, ref_c, ref_pts):
    def prep(queries, points, npad):
        n = points.shape[0]
        q2 = jnp.pad(queries * 2.0, ((0, 0), (0, 5)))
        qq = jnp.sum(queries * queries, axis=1, keepdims=True)
        pt = jnp.pad(points.T, ((0, 5), (0, npad - n)))
        pp = jnp.sum(points * points, axis=1)
        pp = jnp.pad(pp, (0, npad - n), constant_values=jnp.inf)[None, :]
        return q2, qq, pt, pp

    q2s, qqs, pts, pps = prep(src_c, src_pts, _NS * _B)
    q2r, qqr, ptr, ppr = prep(ref_c, ref_pts, _NR * _B)
    q2 = jnp.stack([q2s, q2r])
    qq = jnp.stack([qqs, qqr])
    pt = jnp.concatenate([pts, ptr], axis=1)
    pp = jnp.concatenate([pps, ppr], axis=1)

    cloud = lambda j: jnp.where(j < _NS, 0, 1)
    return pl.pallas_call(
        _nn_body,
        grid=(_NS + _NR,),
        in_specs=[
            pl.BlockSpec((1, _Q, 8), lambda j: (cloud(j), 0, 0)),
            pl.BlockSpec((1, _Q, 1), lambda j: (cloud(j), 0, 0)),
            pl.BlockSpec((8, _B), lambda j: (0, j)),
            pl.BlockSpec((1, _B), lambda j: (0, j)),
        ],
        out_specs=[
            pl.BlockSpec((_Q, 1), lambda j: (0, 0)),
            pl.BlockSpec((_Q, 1), lambda j: (0, 0)),
        ],
        out_shape=[
            jax.ShapeDtypeStruct((_Q, 1), jnp.int32),
            jax.ShapeDtypeStruct((_Q, 1), jnp.int32),
        ],
        scratch_shapes=[
            pltpu.VMEM((_Q, _B), jnp.float32),
            pltpu.VMEM((_Q, _B), jnp.int32),
        ],
    )(q2, qq, pt, pp)


_SC_MESH = dict(core_axis_name="c", subcore_axis_name="s",
                num_cores=2, num_subcores=16)
_BMS = 61440
_BMR = 51200


def _sc_scatter_body(sback, rback, zeros_h, ones_h, bms, bmr,
                     shared, idx_s, ones_s, idx_r, ones_r):
    c = jax.lax.axis_index("c")
    s = jax.lax.axis_index("s")

    @pl.when(c == 0)
    def _src():
        zl = _BMS // 16
        pltpu.sync_copy(zeros_h.at[pl.ds(s * zl, zl)],
                        shared.at[pl.ds(s * zl, zl)])
        pltpu.sync_copy(sback.at[pl.ds(s * 1920, 1920)], idx_s)
        pltpu.sync_copy(ones_h.at[pl.ds(0, 1920)], ones_s)
        plsc.subcore_barrier()
        pltpu.sync_copy(ones_s, shared.at[idx_s])
        plsc.subcore_barrier()
        pltpu.sync_copy(shared.at[pl.ds(s * zl, zl)],
                        bms.at[pl.ds(s * zl, zl)])

    @pl.when(c == 1)
    def _ref():
        zl = _BMR // 16
        pltpu.sync_copy(zeros_h.at[pl.ds(s * zl, zl)],
                        shared.at[pl.ds(s * zl, zl)])
        pltpu.sync_copy(rback.at[pl.ds(s * 1600, 1600)], idx_r)
        pltpu.sync_copy(ones_h.at[pl.ds(0, 1600)], ones_r)
        plsc.subcore_barrier()
        pltpu.sync_copy(ones_r, shared.at[idx_r])
        plsc.subcore_barrier()
        pltpu.sync_copy(shared.at[pl.ds(s * zl, zl)],
                        bmr.at[pl.ds(s * zl, zl)])


def _sc_scatter(src_back, ref_back):
    sback = jnp.pad(src_back, (0, 30720 - src_back.shape[0]),
                    constant_values=_BMS - 1)
    rback = jnp.pad(ref_back, (0, 25600 - ref_back.shape[0]),
                    constant_values=_BMR - 1)
    zeros_h = jnp.zeros((_BMS,), jnp.float32)
    ones_h = jnp.ones((1920,), jnp.float32)
    f = pl.kernel(
        _sc_scatter_body,
        out_type=[
            jax.ShapeDtypeStruct((_BMS,), jnp.float32),
            jax.ShapeDtypeStruct((_BMR,), jnp.float32),
        ],
        mesh=plsc.VectorSubcoreMesh(**_SC_MESH),
        scratch_types=[
            pltpu.VMEM_SHARED((_BMS,), jnp.float32),
            pltpu.VMEM((1920,), jnp.int32),
            pltpu.VMEM((1920,), jnp.float32),
            pltpu.VMEM((1600,), jnp.int32),
            pltpu.VMEM((1600,), jnp.float32),
        ],
    )
    return f(sback, rback, zeros_h, ones_h)


def _sc_gather_body(bms, bmr, idxs, idxr, masks, maskr, qi, qv, sem):
    c = jax.lax.axis_index("c")
    s = jax.lax.axis_index("s")
    w = s * 2 + c
    pltpu.sync_copy(idxs.at[pl.ds(w * 32, 32)], qi)
    pltpu.async_copy(bms.at[qi], qv, sem).wait()
    pltpu.sync_copy(qv, masks.at[pl.ds(w * 32, 32)])
    pltpu.sync_copy(idxr.at[pl.ds(w * 32, 32)], qi)
    pltpu.async_copy(bmr.at[qi], qv, sem).wait()
    pltpu.sync_copy(qv, maskr.at[pl.ds(w * 32, 32)])


def _sc_gather(bm_src, bm_ref, idx_src, idx_ref):
    f = pl.kernel(
        _sc_gather_body,
        out_type=[
            jax.ShapeDtypeStruct((_Q,), jnp.float32),
            jax.ShapeDtypeStruct((_Q,), jnp.float32),
        ],
        mesh=plsc.VectorSubcoreMesh(**_SC_MESH),
        scratch_types=[
            pltpu.VMEM((32,), jnp.int32),
            pltpu.VMEM((32,), jnp.float32),
            pltpu.SemaphoreType.DMA,
        ],
    )
    return f(bm_src, bm_ref, idx_src, idx_ref)


def _isin_body(idx_ref, back_ref, outs_ref, outr_ref):
    j = pl.program_id(0)

    @pl.when(j == 0)
    def _init_s():
        outs_ref[...] = jnp.zeros(outs_ref.shape, jnp.float32)

    @pl.when(j == _MS)
    def _init_r():
        outr_ref[...] = jnp.zeros(outr_ref.shape, jnp.float32)

    eq = idx_ref[0] == back_ref[...]
    hit = jnp.max(jnp.where(eq, 1.0, 0.0), axis=1, keepdims=True)

    @pl.when(j < _MS)
    def _acc_s():
        outs_ref[...] = jnp.maximum(outs_ref[...], hit)

    @pl.when(j >= _MS)
    def _acc_r():
        outr_ref[...] = jnp.maximum(outr_ref[...], hit)


def _isin(idx_src, src_back, idx_ref, ref_back):
    backs = jnp.pad(src_back, (0, _MS * _BB - src_back.shape[0]),
                    constant_values=-1)
    backr = jnp.pad(ref_back, (0, _MR * _BB - ref_back.shape[0]),
                    constant_values=-1)
    back = jnp.concatenate([backs, backr])[None, :]
    idx = jnp.stack([idx_src, idx_ref])
    cloud = lambda j: jnp.where(j < _MS, 0, 1)
    return pl.pallas_call(
        _isin_body,
        grid=(_MS + _MR,),
        in_specs=[
            pl.BlockSpec((1, _Q, 1), lambda j: (cloud(j), 0, 0)),
            pl.BlockSpec((1, _BB), lambda j: (0, j)),
        ],
        out_specs=[
            pl.BlockSpec((_Q, 1), lambda j: (0, 0)),
            pl.BlockSpec((_Q, 1), lambda j: (0, 0)),
        ],
        out_shape=[
            jax.ShapeDtypeStruct((_Q, 1), jnp.float32),
            jax.ShapeDtypeStruct((_Q, 1), jnp.float32),
        ],
    )(idx, back)


def _pairs_body(mr_r, mr_c, ms_r, ms_c, refn_c, refn_r, srcn_c, srcn_r,
                gx_c, gx_r, gy_c, gy_r, ov_c, ov_r,
                loss_ref, loss1_ref, loss2_ref, gtm_ref):
    f32 = jnp.float32
    keC = refn_c[...] * 1024 + srcn_c[...]
    keR = refn_r[...] * 1024 + srcn_r[...]
    eqE = keC == keR
    iE_r = jax.lax.broadcasted_iota(jnp.int32, (512, 512), 0)
    iE_c = jax.lax.broadcasted_iota(jnp.int32, (512, 512), 1)
    dupE = jnp.any(eqE & (iE_c < iE_r), axis=1, keepdims=True)
    n_pos = jnp.sum(jnp.where(dupE, f32(0), f32(1)))

    lane_g = jax.lax.broadcasted_iota(jnp.int32, (2048, 1024), 1)
    mrg = jnp.sum(jnp.where(lane_g == gx_c[...], mr_r[...], f32(0)),
                  axis=1, keepdims=True)
    msg = jnp.sum(jnp.where(lane_g == gy_c[...], ms_r[...], f32(0)),
                  axis=1, keepdims=True)
    mC = (mrg * msg) > 0
    sub_g = jax.lax.broadcasted_iota(jnp.int32, (1024, 2048), 0)
    mrgR = jnp.sum(jnp.where(sub_g == gx_r[...], mr_c[...], f32(0)),
                   axis=0, keepdims=True)
    msgR = jnp.sum(jnp.where(sub_g == gy_r[...], ms_c[...], f32(0)),
                   axis=0, keepdims=True)
    mR = (mrgR * msgR) > 0

    kgC = gx_c[...] * 1024 + gy_c[...]
    kgR = gx_r[...] * 1024 + gy_r[...]
    eqG = kgC == kgR
    iG_r = jax.lax.broadcasted_iota(jnp.int32, (2048, 2048), 0)
    iG_c = jax.lax.broadcasted_iota(jnp.int32, (2048, 2048), 1)
    lower = eqG & (iG_c < iG_r)
    upper = eqG & (iG_c > iG_r)
    firstC = ~jnp.any(lower, axis=1, keepdims=True)
    lastC = ~jnp.any(upper, axis=1, keepdims=True)
    firstR = ~jnp.any(upper, axis=0, keepdims=True)
    lastR = ~jnp.any(lower, axis=0, keepdims=True)
    activeC = firstC & mC
    activeR = firstR & mR
    actF = jnp.where(activeC, f32(1), f32(0))
    n_active = jnp.sum(actF)
    ovl_r = jnp.where(lastR, ov_r[...], f32(0))
    ovl_c = jnp.where(lastC, ov_c[...], f32(0))
    vC = jnp.sum(jnp.where(eqG, ovl_r, f32(0)),
                 axis=1, keepdims=True)
    vR = jnp.sum(jnp.where(eqG, ovl_c, f32(0)),
                 axis=0, keepdims=True)

    beats = activeR & ((vR > vC) | ((vR == vC) & (kgR < kgC)))
    rank = jnp.sum(jnp.where(beats, f32(1), f32(0)), axis=1, keepdims=True)
    selF = jnp.where(activeC & (rank < 256.0), f32(1), f32(0))
    use_topk = n_active > 256.0
    gsel = jnp.where(use_topk, selF, actF)
    n_gt = jnp.sum(gsel)
    eshit = jnp.any(kgC == keR, axis=1, keepdims=True)
    n_both = jnp.sum(jnp.where(eshit, gsel, f32(0)))

    nm = f32(1024 * 1024)
    n_pos_c = jnp.maximum(n_pos, f32(1))
    ratio = 1.0 / (n_pos_c / nm)
    sum_abs = n_gt + n_pos - 2.0 * n_both
    loss1 = f32(math.sqrt(2)) * ratio * (sum_abs / nm)
    loss1_ref[0, 0] = loss1
    loss2_ref[0, 0] = f32(0)
    loss_ref[0, 0] = loss1 + f32(0)
    gtm_ref[...] = 1.0 - jnp.concatenate([mr_r[...], ms_r[...]], axis=1)


def _pairs(mask_ref_v, mask_src_v, refn, srcn, gx, gy, ov):
    args = (mask_ref_v.reshape(1, 1024), mask_ref_v.reshape(1024, 1),
            mask_src_v.reshape(1, 1024), mask_src_v.reshape(1024, 1),
            refn.reshape(512, 1), refn.reshape(1, 512),
            srcn.reshape(512, 1), srcn.reshape(1, 512),
            gx.reshape(2048, 1), gx.reshape(1, 2048),
            gy.reshape(2048, 1), gy.reshape(1, 2048),
            ov.reshape(2048, 1), ov.reshape(1, 2048))
    specs = [pl.BlockSpec(a.shape, lambda: (0, 0)) for a in args]
    return pl.pallas_call(
        _pairs_body,
        in_specs=specs,
        out_specs=[
            pl.BlockSpec(memory_space=pltpu.SMEM),
            pl.BlockSpec(memory_space=pltpu.SMEM),
            pl.BlockSpec(memory_space=pltpu.SMEM),
            pl.BlockSpec((1, 2048), lambda: (0, 0)),
        ],
        out_shape=[
            jax.ShapeDtypeStruct((1, 1), jnp.float32),
            jax.ShapeDtypeStruct((1, 1), jnp.float32),
            jax.ShapeDtypeStruct((1, 1), jnp.float32),
            jax.ShapeDtypeStruct((1, 2048), jnp.float32),
        ],
    )(*args)


def kernel(src_points, ref_points, src_points_c, ref_points_c,
           src_node_corr_indices, ref_node_corr_indices,
           gt_node_corr_indices, gt_node_corr_overlaps,
           src_back_indices, ref_back_indices):
    bm_src, bm_ref = _sc_scatter(src_back_indices, ref_back_indices)
    idx_src, idx_ref = _nn_argmin(src_points_c, src_points,
                                  ref_points_c, ref_points)
    mask_src, mask_ref = _sc_gather(bm_src, bm_ref,
                                    idx_src.reshape(-1), idx_ref.reshape(-1))
    loss, loss1, loss2, inv_gtm = _pairs(
        mask_ref.reshape(-1), mask_src.reshape(-1),
        ref_node_corr_indices, src_node_corr_indices,
        gt_node_corr_indices[:, 0], gt_node_corr_indices[:, 1],
        gt_node_corr_overlaps)
    return (loss.reshape(()), loss1.reshape(()), loss2.reshape(()),
            inv_gtm.reshape(2048))


# speedup vs baseline: 9.6096x; 1.0117x over previous
import functools
import math

import jax
import jax.numpy as jnp
from jax.experimental import pallas as pl
from jax.experimental.pallas import tpu as pltpu
from jax.experimental.pallas import tpu_sc as plsc

_Q = 1024
_B = 2048
_BB = 2048
_NS = 30
_NR = 25
_MS = 15
_MR = 13


def _nn_body(q2_ref, qq_ref, pt_ref, pp_ref, outs_ref, outr_ref, rm_ref, ri_ref):
    j = pl.program_id(0)

    @pl.when((j == 0) | (j == _NS))
    def _init():
        rm_ref[...] = jnp.full(rm_ref.shape, jnp.inf, jnp.float32)
        ri_ref[...] = jnp.zeros(ri_ref.shape, jnp.int32)

    mm2 = jax.lax.dot_general(
        q2_ref[0], pt_ref[...], (((1,), (0,)), ((), ())),
        preferred_element_type=jnp.float32)
    t = qq_ref[0] + pp_ref[...]
    d2 = t - mm2
    rm = rm_ref[...]
    lt = d2 < rm
    rm_ref[...] = jnp.minimum(d2, rm)
    ri_ref[...] = jnp.where(lt, j, ri_ref[...])

    def _argmin(base):
        rmf = rm_ref[...]
        pos = jax.lax.broadcasted_iota(jnp.int32, (_Q, _B), 1)
        gidx = (ri_ref[...] - base) * _B + pos
        best = jnp.min(rmf, axis=1, keepdims=True)
        ii = jnp.where(rmf == best, gidx, jnp.int32(2**30))
        return jnp.min(ii, axis=1, keepdims=True)

    @pl.when(j == _NS - 1)
    def _fin_src():
        outs_ref[...] = _argmin(0)

    @pl.when(j == _NS + _NR - 1)
    def _fin_ref():
        outr_ref[...] = _argmin(_NS)


def _nn_argmin(src_c, src_pts, ref_c, ref_pts):
    def prep(queries, points, npad):
        n = points.shape[0]
        q2 = jnp.pad(queries * 2.0, ((0, 0), (0, 5)))
        qq = jnp.sum(queries * queries, axis=1, keepdims=True)
        pt = jnp.pad(points.T, ((0, 5), (0, npad - n)))
        pp = jnp.sum(points * points, axis=1)
        pp = jnp.pad(pp, (0, npad - n), constant_values=jnp.inf)[None, :]
        return q2, qq, pt, pp

    q2s, qqs, pts, pps = prep(src_c, src_pts, _NS * _B)
    q2r, qqr, ptr, ppr = prep(ref_c, ref_pts, _NR * _B)
    q2 = jnp.stack([q2s, q2r])
    qq = jnp.stack([qqs, qqr])
    pt = jnp.concatenate([pts, ptr], axis=1)
    pp = jnp.concatenate([pps, ppr], axis=1)

    cloud = lambda j: jnp.where(j < _NS, 0, 1)
    return pl.pallas_call(
        _nn_body,
        grid=(_NS + _NR,),
        in_specs=[
            pl.BlockSpec((1, _Q, 8), lambda j: (cloud(j), 0, 0)),
            pl.BlockSpec((1, _Q, 1), lambda j: (cloud(j), 0, 0)),
            pl.BlockSpec((8, _B), lambda j: (0, j)),
            pl.BlockSpec((1, _B), lambda j: (0, j)),
        ],
        out_specs=[
            pl.BlockSpec((_Q, 1), lambda j: (0, 0)),
            pl.BlockSpec((_Q, 1), lambda j: (0, 0)),
        ],
        out_shape=[
            jax.ShapeDtypeStruct((_Q, 1), jnp.int32),
            jax.ShapeDtypeStruct((_Q, 1), jnp.int32),
        ],
        scratch_shapes=[
            pltpu.VMEM((_Q, _B), jnp.float32),
            pltpu.VMEM((_Q, _B), jnp.int32),
        ],
    )(q2, qq, pt, pp)


_SC_MESH = dict(core_axis_name="c", subcore_axis_name="s",
                num_cores=2, num_subcores=16)
_BMS = 61440
_BMR = 51200


def _sc_scatter_body(sback, rback, zeros_h, ones_h, bms, bmr,
                     shared, idx_s, ones_s, idx_r, ones_r):
    c = jax.lax.axis_index("c")
    s = jax.lax.axis_index("s")

    @pl.when(c == 0)
    def _src():
        zl = _BMS // 16
        pltpu.sync_copy(zeros_h.at[pl.ds(s * zl, zl)],
                        shared.at[pl.ds(s * zl, zl)])
        pltpu.sync_copy(sback.at[pl.ds(s * 1920, 1920)], idx_s)
        pltpu.sync_copy(ones_h.at[pl.ds(0, 1920)], ones_s)
        plsc.subcore_barrier()
        pltpu.sync_copy(ones_s, shared.at[idx_s])
        plsc.subcore_barrier()
        pltpu.sync_copy(shared.at[pl.ds(s * zl, zl)],
                        bms.at[pl.ds(s * zl, zl)])

    @pl.when(c == 1)
    def _ref():
        zl = _BMR // 16
        pltpu.sync_copy(zeros_h.at[pl.ds(s * zl, zl)],
                        shared.at[pl.ds(s * zl, zl)])
        pltpu.sync_copy(rback.at[pl.ds(s * 1600, 1600)], idx_r)
        pltpu.sync_copy(ones_h.at[pl.ds(0, 1600)], ones_r)
        plsc.subcore_barrier()
        pltpu.sync_copy(ones_r, shared.at[idx_r])
        plsc.subcore_barrier()
        pltpu.sync_copy(shared.at[pl.ds(s * zl, zl)],
                        bmr.at[pl.ds(s * zl, zl)])


def _sc_scatter(src_back, ref_back):
    sback = jnp.pad(src_back, (0, 30720 - src_back.shape[0]),
                    constant_values=_BMS - 1)
    rback = jnp.pad(ref_back, (0, 25600 - ref_back.shape[0]),
                    constant_values=_BMR - 1)
    zeros_h = jnp.zeros((_BMS,), jnp.float32)
    ones_h = jnp.ones((1920,), jnp.float32)
    f = pl.kernel(
        _sc_scatter_body,
        out_type=[
            jax.ShapeDtypeStruct((_BMS,), jnp.float32),
            jax.ShapeDtypeStruct((_BMR,), jnp.float32),
        ],
        mesh=plsc.VectorSubcoreMesh(**_SC_MESH),
        scratch_types=[
            pltpu.VMEM_SHARED((_BMS,), jnp.float32),
            pltpu.VMEM((1920,), jnp.int32),
            pltpu.VMEM((1920,), jnp.float32),
            pltpu.VMEM((1600,), jnp.int32),
            pltpu.VMEM((1600,), jnp.float32),
        ],
    )
    return f(sback, rback, zeros_h, ones_h)


def _sc_gather_body(bms, bmr, idxs, idxr, gx, gy, masks, maskr, mgx, mgy,
                    qi, qv, gi, gidx, gv, sem):
    c = jax.lax.axis_index("c")
    s = jax.lax.axis_index("s")
    w = s * 2 + c
    pltpu.sync_copy(idxs.at[pl.ds(w * 32, 32)], qi)
    pltpu.async_copy(bms.at[qi], qv, sem).wait()
    pltpu.sync_copy(qv, masks.at[pl.ds(w * 32, 32)])
    pltpu.sync_copy(idxr.at[pl.ds(w * 32, 32)], qi)
    pltpu.async_copy(bmr.at[qi], qv, sem).wait()
    pltpu.sync_copy(qv, maskr.at[pl.ds(w * 32, 32)])
    pltpu.sync_copy(gx.at[pl.ds(w * 64, 64)], gi)
    pltpu.async_copy(idxr.at[gi], gidx, sem).wait()
    pltpu.async_copy(bmr.at[gidx], gv, sem).wait()
    pltpu.sync_copy(gv, mgx.at[pl.ds(w * 64, 64)])
    pltpu.sync_copy(gy.at[pl.ds(w * 64, 64)], gi)
    pltpu.async_copy(idxs.at[gi], gidx, sem).wait()
    pltpu.async_copy(bms.at[gidx], gv, sem).wait()
    pltpu.sync_copy(gv, mgy.at[pl.ds(w * 64, 64)])


def _sc_gather(bm_src, bm_ref, idx_src, idx_ref, gx, gy):
    f = pl.kernel(
        _sc_gather_body,
        out_type=[
            jax.ShapeDtypeStruct((_Q,), jnp.float32),
            jax.ShapeDtypeStruct((_Q,), jnp.float32),
            jax.ShapeDtypeStruct((2048,), jnp.float32),
            jax.ShapeDtypeStruct((2048,), jnp.float32),
        ],
        mesh=plsc.VectorSubcoreMesh(**_SC_MESH),
        scratch_types=[
            pltpu.VMEM((32,), jnp.int32),
            pltpu.VMEM((32,), jnp.float32),
            pltpu.VMEM((64,), jnp.int32),
            pltpu.VMEM((64,), jnp.int32),
            pltpu.VMEM((64,), jnp.float32),
            pltpu.SemaphoreType.DMA,
        ],
    )
    return f(bm_src, bm_ref, idx_src, idx_ref, gx, gy)


def _isin_body(idx_ref, back_ref, outs_ref, outr_ref):
    j = pl.program_id(0)

    @pl.when(j == 0)
    def _init_s():
        outs_ref[...] = jnp.zeros(outs_ref.shape, jnp.float32)

    @pl.when(j == _MS)
    def _init_r():
        outr_ref[...] = jnp.zeros(outr_ref.shape, jnp.float32)

    eq = idx_ref[0] == back_ref[...]
    hit = jnp.max(jnp.where(eq, 1.0, 0.0), axis=1, keepdims=True)

    @pl.when(j < _MS)
    def _acc_s():
        outs_ref[...] = jnp.maximum(outs_ref[...], hit)

    @pl.when(j >= _MS)
    def _acc_r():
        outr_ref[...] = jnp.maximum(outr_ref[...], hit)


def _isin(idx_src, src_back, idx_ref, ref_back):
    backs = jnp.pad(src_back, (0, _MS * _BB - src_back.shape[0]),
                    constant_values=-1)
    backr = jnp.pad(ref_back, (0, _MR * _BB - ref_back.shape[0]),
                    constant_values=-1)
    back = jnp.concatenate([backs, backr])[None, :]
    idx = jnp.stack([idx_src, idx_ref])
    cloud = lambda j: jnp.where(j < _MS, 0, 1)
    return pl.pallas_call(
        _isin_body,
        grid=(_MS + _MR,),
        in_specs=[
            pl.BlockSpec((1, _Q, 1), lambda j: (cloud(j), 0, 0)),
            pl.BlockSpec((1, _BB), lambda j: (0, j)),
        ],
        out_specs=[
            pl.BlockSpec((_Q, 1), lambda j: (0, 0)),
            pl.BlockSpec((_Q, 1), lambda j: (0, 0)),
        ],
        out_shape=[
            jax.ShapeDtypeStruct((_Q, 1), jnp.float32),
            jax.ShapeDtypeStruct((_Q, 1), jnp.float32),
        ],
    )(idx, back)


def _pairs_body(mr_r, ms_r, mg_c, mg_r, refn_c, refn_r, srcn_c, srcn_r,
                gx_c, gx_r, gy_c, gy_r, ov_c, ov_r,
                loss_ref, loss1_ref, loss2_ref, gtm_ref):
    f32 = jnp.float32
    keC = refn_c[...] * 1024 + srcn_c[...]
    keR = refn_r[...] * 1024 + srcn_r[...]
    eqE = keC == keR
    iE_r = jax.lax.broadcasted_iota(jnp.int32, (512, 512), 0)
    iE_c = jax.lax.broadcasted_iota(jnp.int32, (512, 512), 1)
    dupE = jnp.any(eqE & (iE_c < iE_r), axis=1, keepdims=True)
    n_pos = jnp.sum(jnp.where(dupE, f32(0), f32(1)))

    mC = mg_c[...] > 0
    mR = mg_r[...] > 0

    kgC = gx_c[...] * 1024 + gy_c[...]
    kgR = gx_r[...] * 1024 + gy_r[...]
    eqG = kgC == kgR
    iG_r = jax.lax.broadcasted_iota(jnp.int32, (2048, 2048), 0)
    iG_c = jax.lax.broadcasted_iota(jnp.int32, (2048, 2048), 1)
    lower = eqG & (iG_c < iG_r)
    upper = eqG & (iG_c > iG_r)
    firstC = ~jnp.any(lower, axis=1, keepdims=True)
    lastC = ~jnp.any(upper, axis=1, keepdims=True)
    firstR = ~jnp.any(upper, axis=0, keepdims=True)
    lastR = ~jnp.any(lower, axis=0, keepdims=True)
    activeC = firstC & mC
    activeR = firstR & mR
    actF = jnp.where(activeC, f32(1), f32(0))
    n_active = jnp.sum(actF)
    ovl_r = jnp.where(lastR, ov_r[...], f32(0))
    ovl_c = jnp.where(lastC, ov_c[...], f32(0))
    vC = jnp.sum(jnp.where(eqG, ovl_r, f32(0)),
                 axis=1, keepdims=True)
    vR = jnp.sum(jnp.where(eqG, ovl_c, f32(0)),
                 axis=0, keepdims=True)

    beats = activeR & ((vR > vC) | ((vR == vC) & (kgR < kgC)))
    rank = jnp.sum(jnp.where(beats, f32(1), f32(0)), axis=1, keepdims=True)
    selF = jnp.where(activeC & (rank < 256.0), f32(1), f32(0))
    use_topk = n_active > 256.0
    gsel = jnp.where(use_topk, selF, actF)
    n_gt = jnp.sum(gsel)
    eshit = jnp.any(kgC == keR, axis=1, keepdims=True)
    n_both = jnp.sum(jnp.where(eshit, gsel, f32(0)))

    nm = f32(1024 * 1024)
    n_pos_c = jnp.maximum(n_pos, f32(1))
    ratio = 1.0 / (n_pos_c / nm)
    sum_abs = n_gt + n_pos - 2.0 * n_both
    loss1 = f32(math.sqrt(2)) * ratio * (sum_abs / nm)
    loss1_ref[0, 0] = loss1
    loss2_ref[0, 0] = f32(0)
    loss_ref[0, 0] = loss1 + f32(0)
    gtm_ref[...] = 1.0 - jnp.concatenate([mr_r[...], ms_r[...]], axis=1)


def _pairs(mask_ref_v, mask_src_v, mg, refn, srcn, gx, gy, ov):
    args = (mask_ref_v.reshape(1, 1024), mask_src_v.reshape(1, 1024),
            mg.reshape(2048, 1), mg.reshape(1, 2048),
            refn.reshape(512, 1), refn.reshape(1, 512),
            srcn.reshape(512, 1), srcn.reshape(1, 512),
            gx.reshape(2048, 1), gx.reshape(1, 2048),
            gy.reshape(2048, 1), gy.reshape(1, 2048),
            ov.reshape(2048, 1), ov.reshape(1, 2048))
    specs = [pl.BlockSpec(a.shape, lambda: (0, 0)) for a in args]
    return pl.pallas_call(
        _pairs_body,
        in_specs=specs,
        out_specs=[
            pl.BlockSpec(memory_space=pltpu.SMEM),
            pl.BlockSpec(memory_space=pltpu.SMEM),
            pl.BlockSpec(memory_space=pltpu.SMEM),
            pl.BlockSpec((1, 2048), lambda: (0, 0)),
        ],
        out_shape=[
            jax.ShapeDtypeStruct((1, 1), jnp.float32),
            jax.ShapeDtypeStruct((1, 1), jnp.float32),
            jax.ShapeDtypeStruct((1, 1), jnp.float32),
            jax.ShapeDtypeStruct((1, 2048), jnp.float32),
        ],
    )(*args)


def kernel(src_points, ref_points, src_points_c, ref_points_c,
           src_node_corr_indices, ref_node_corr_indices,
           gt_node_corr_indices, gt_node_corr_overlaps,
           src_back_indices, ref_back_indices):
    bm_src, bm_ref = _sc_scatter(src_back_indices, ref_back_indices)
    idx_src, idx_ref = _nn_argmin(src_points_c, src_points,
                                  ref_points_c, ref_points)
    gx = gt_node_corr_indices[:, 0]
    gy = gt_node_corr_indices[:, 1]
    mask_src, mask_ref, mgx, mgy = _sc_gather(
        bm_src, bm_ref, idx_src.reshape(-1), idx_ref.reshape(-1), gx, gy)
    loss, loss1, loss2, inv_gtm = _pairs(
        mask_ref.reshape(-1), mask_src.reshape(-1), mgx * mgy,
        ref_node_corr_indices, src_node_corr_indices,
        gx, gy, gt_node_corr_overlaps)
    return (loss.reshape(()), loss1.reshape(()), loss2.reshape(()),
            inv_gtm.reshape(2048))


# speedup vs baseline: 9.6144x; 1.0005x over previous
import math

import jax
import jax.numpy as jnp
from jax.experimental import pallas as pl
from jax.experimental.pallas import tpu as pltpu
from jax.experimental.pallas import tpu_sc as plsc

_Q = 1024
_B = 2048
_NS = 30
_NR = 25


def _nn_body(q2_ref, qq_ref, pt_ref, pp_ref, outs_ref, outr_ref, rm_ref, ri_ref):
    j = pl.program_id(0)

    @pl.when((j == 0) | (j == _NS))
    def _init():
        rm_ref[...] = jnp.full(rm_ref.shape, jnp.inf, jnp.float32)
        ri_ref[...] = jnp.zeros(ri_ref.shape, jnp.int32)

    mm2 = jax.lax.dot_general(
        q2_ref[0], pt_ref[...], (((1,), (0,)), ((), ())),
        preferred_element_type=jnp.float32)
    t = qq_ref[0] + pp_ref[...]
    d2 = t - mm2
    rm = rm_ref[...]
    lt = d2 < rm
    rm_ref[...] = jnp.minimum(d2, rm)
    ri_ref[...] = jnp.where(lt, j, ri_ref[...])

    def _argmin(base):
        rmf = rm_ref[...]
        pos = jax.lax.broadcasted_iota(jnp.int32, (_Q, _B), 1)
        gidx = (ri_ref[...] - base) * _B + pos
        best = jnp.min(rmf, axis=1, keepdims=True)
        ii = jnp.where(rmf == best, gidx, jnp.int32(2**30))
        return jnp.min(ii, axis=1, keepdims=True)

    @pl.when(j == _NS - 1)
    def _fin_src():
        outs_ref[...] = _argmin(0)

    @pl.when(j == _NS + _NR - 1)
    def _fin_ref():
        outr_ref[...] = _argmin(_NS)


def _nn_argmin(src_c, src_pts, ref_c, ref_pts):
    def prep(queries, points, npad):
        n = points.shape[0]
        q2 = jnp.pad(queries * 2.0, ((0, 0), (0, 5)))
        qq = jnp.sum(queries * queries, axis=1, keepdims=True)
        pt = jnp.pad(points.T, ((0, 5), (0, npad - n)))
        pp = jnp.sum(points * points, axis=1)
        pp = jnp.pad(pp, (0, npad - n), constant_values=jnp.inf)[None, :]
        return q2, qq, pt, pp

    q2s, qqs, pts, pps = prep(src_c, src_pts, _NS * _B)
    q2r, qqr, ptr, ppr = prep(ref_c, ref_pts, _NR * _B)
    q2 = jnp.stack([q2s, q2r])
    qq = jnp.stack([qqs, qqr])
    pt = jnp.concatenate([pts, ptr], axis=1)
    pp = jnp.concatenate([pps, ppr], axis=1)

    cloud = lambda j: jnp.where(j < _NS, 0, 1)
    return pl.pallas_call(
        _nn_body,
        grid=(_NS + _NR,),
        in_specs=[
            pl.BlockSpec((1, _Q, 8), lambda j: (cloud(j), 0, 0)),
            pl.BlockSpec((1, _Q, 1), lambda j: (cloud(j), 0, 0)),
            pl.BlockSpec((8, _B), lambda j: (0, j)),
            pl.BlockSpec((1, _B), lambda j: (0, j)),
        ],
        out_specs=[
            pl.BlockSpec((_Q, 1), lambda j: (0, 0)),
            pl.BlockSpec((_Q, 1), lambda j: (0, 0)),
        ],
        out_shape=[
            jax.ShapeDtypeStruct((_Q, 1), jnp.int32),
            jax.ShapeDtypeStruct((_Q, 1), jnp.int32),
        ],
        scratch_shapes=[
            pltpu.VMEM((_Q, _B), jnp.float32),
            pltpu.VMEM((_Q, _B), jnp.int32),
        ],
    )(q2, qq, pt, pp)


_SC_MESH = dict(core_axis_name="c", subcore_axis_name="s",
                num_cores=2, num_subcores=16)
_BMS = 61440
_BMR = 51200


def _sc_scatter_body(sback, rback, zeros_h, ones_h, bms, bmr,
                     shared, idx_s, ones_s, idx_r, ones_r):
    c = jax.lax.axis_index("c")
    s = jax.lax.axis_index("s")

    @pl.when(c == 0)
    def _src():
        zl = _BMS // 16
        pltpu.sync_copy(zeros_h.at[pl.ds(s * zl, zl)],
                        shared.at[pl.ds(s * zl, zl)])
        pltpu.sync_copy(sback.at[pl.ds(s * 1920, 1920)], idx_s)
        pltpu.sync_copy(ones_h.at[pl.ds(0, 1920)], ones_s)
        plsc.subcore_barrier()
        pltpu.sync_copy(ones_s, shared.at[idx_s])
        plsc.subcore_barrier()
        pltpu.sync_copy(shared.at[pl.ds(s * zl, zl)],
                        bms.at[pl.ds(s * zl, zl)])

    @pl.when(c == 1)
    def _ref():
        zl = _BMR // 16
        pltpu.sync_copy(zeros_h.at[pl.ds(s * zl, zl)],
                        shared.at[pl.ds(s * zl, zl)])
        pltpu.sync_copy(rback.at[pl.ds(s * 1600, 1600)], idx_r)
        pltpu.sync_copy(ones_h.at[pl.ds(0, 1600)], ones_r)
        plsc.subcore_barrier()
        pltpu.sync_copy(ones_r, shared.at[idx_r])
        plsc.subcore_barrier()
        pltpu.sync_copy(shared.at[pl.ds(s * zl, zl)],
                        bmr.at[pl.ds(s * zl, zl)])


def _sc_scatter(src_back, ref_back):
    sback = jnp.pad(src_back, (0, 30720 - src_back.shape[0]),
                    constant_values=_BMS - 1)
    rback = jnp.pad(ref_back, (0, 25600 - ref_back.shape[0]),
                    constant_values=_BMR - 1)
    zeros_h = jnp.zeros((_BMS,), jnp.float32)
    ones_h = jnp.ones((1920,), jnp.float32)
    f = pl.kernel(
        _sc_scatter_body,
        out_type=[
            jax.ShapeDtypeStruct((_BMS,), jnp.float32),
            jax.ShapeDtypeStruct((_BMR,), jnp.float32),
        ],
        mesh=plsc.VectorSubcoreMesh(**_SC_MESH),
        scratch_types=[
            pltpu.VMEM_SHARED((_BMS,), jnp.float32),
            pltpu.VMEM((1920,), jnp.int32),
            pltpu.VMEM((1920,), jnp.float32),
            pltpu.VMEM((1600,), jnp.int32),
            pltpu.VMEM((1600,), jnp.float32),
        ],
    )
    return f(sback, rback, zeros_h, ones_h)


def _sc_gather_body(bms, bmr, idxs, idxr, gx, gy, masks, maskr, mgx, mgy,
                    qi, qv, gi, gidx, gv, sem):
    c = jax.lax.axis_index("c")
    s = jax.lax.axis_index("s")
    w = s * 2 + c
    pltpu.sync_copy(idxs.at[pl.ds(w * 32, 32)], qi)
    pltpu.async_copy(bms.at[qi], qv, sem).wait()
    pltpu.sync_copy(qv, masks.at[pl.ds(w * 32, 32)])
    pltpu.sync_copy(idxr.at[pl.ds(w * 32, 32)], qi)
    pltpu.async_copy(bmr.at[qi], qv, sem).wait()
    pltpu.sync_copy(qv, maskr.at[pl.ds(w * 32, 32)])
    pltpu.sync_copy(gx.at[pl.ds(w * 64, 64)], gi)
    pltpu.async_copy(idxr.at[gi], gidx, sem).wait()
    pltpu.async_copy(bmr.at[gidx], gv, sem).wait()
    pltpu.sync_copy(gv, mgx.at[pl.ds(w * 64, 64)])
    pltpu.sync_copy(gy.at[pl.ds(w * 64, 64)], gi)
    pltpu.async_copy(idxs.at[gi], gidx, sem).wait()
    pltpu.async_copy(bms.at[gidx], gv, sem).wait()
    pltpu.sync_copy(gv, mgy.at[pl.ds(w * 64, 64)])


def _sc_gather(bm_src, bm_ref, idx_src, idx_ref, gx, gy):
    f = pl.kernel(
        _sc_gather_body,
        out_type=[
            jax.ShapeDtypeStruct((_Q,), jnp.float32),
            jax.ShapeDtypeStruct((_Q,), jnp.float32),
            jax.ShapeDtypeStruct((2048,), jnp.float32),
            jax.ShapeDtypeStruct((2048,), jnp.float32),
        ],
        mesh=plsc.VectorSubcoreMesh(**_SC_MESH),
        scratch_types=[
            pltpu.VMEM((32,), jnp.int32),
            pltpu.VMEM((32,), jnp.float32),
            pltpu.VMEM((64,), jnp.int32),
            pltpu.VMEM((64,), jnp.int32),
            pltpu.VMEM((64,), jnp.float32),
            pltpu.SemaphoreType.DMA,
        ],
    )
    return f(bm_src, bm_ref, idx_src, idx_ref, gx, gy)


def _pairs_body(mr_r, ms_r, mg_c, mg_r, refn_c, refn_r, srcn_c, srcn_r,
                gx_c, gx_r, gy_c, gy_r, ov_c, ov_r,
                loss_ref, loss1_ref, loss2_ref, gtm_ref):
    f32 = jnp.float32
    keC = refn_c[...] * 1024 + srcn_c[...]
    keR = refn_r[...] * 1024 + srcn_r[...]
    eqE = keC == keR
    iE_r = jax.lax.broadcasted_iota(jnp.int32, (512, 512), 0)
    iE_c = jax.lax.broadcasted_iota(jnp.int32, (512, 512), 1)
    dupE = jnp.any(eqE & (iE_c < iE_r), axis=1, keepdims=True)
    n_pos = jnp.sum(jnp.where(dupE, f32(0), f32(1)))

    mC = mg_c[...] > 0
    mR = mg_r[...] > 0

    kgC = gx_c[...] * 1024 + gy_c[...]
    kgR = gx_r[...] * 1024 + gy_r[...]
    eqG = kgC == kgR
    iG_r = jax.lax.broadcasted_iota(jnp.int32, (2048, 2048), 0)
    iG_c = jax.lax.broadcasted_iota(jnp.int32, (2048, 2048), 1)
    lower = eqG & (iG_c < iG_r)
    upper = eqG & (iG_c > iG_r)
    firstC = ~jnp.any(lower, axis=1, keepdims=True)
    lastC = ~jnp.any(upper, axis=1, keepdims=True)
    firstR = ~jnp.any(upper, axis=0, keepdims=True)
    lastR = ~jnp.any(lower, axis=0, keepdims=True)
    activeC = firstC & mC
    activeR = firstR & mR
    actF = jnp.where(activeC, f32(1), f32(0))
    n_active = jnp.sum(actF)
    ovl_r = jnp.where(lastR, ov_r[...], f32(0))
    ovl_c = jnp.where(lastC, ov_c[...], f32(0))
    vC = jnp.sum(jnp.where(eqG, ovl_r, f32(0)),
                 axis=1, keepdims=True)
    vR = jnp.sum(jnp.where(eqG, ovl_c, f32(0)),
                 axis=0, keepdims=True)

    beats = activeR & ((vR > vC) | ((vR == vC) & (kgR < kgC)))
    rank = jnp.sum(jnp.where(beats, f32(1), f32(0)), axis=1, keepdims=True)
    selF = jnp.where(activeC & (rank < 256.0), f32(1), f32(0))
    use_topk = n_active > 256.0
    gsel = jnp.where(use_topk, selF, actF)
    n_gt = jnp.sum(gsel)
    eshit = jnp.any(kgC == keR, axis=1, keepdims=True)
    n_both = jnp.sum(jnp.where(eshit, gsel, f32(0)))

    nm = f32(1024 * 1024)
    n_pos_c = jnp.maximum(n_pos, f32(1))
    ratio = 1.0 / (n_pos_c / nm)
    sum_abs = n_gt + n_pos - 2.0 * n_both
    loss1 = f32(math.sqrt(2)) * ratio * (sum_abs / nm)
    loss1_ref[0, 0] = loss1
    loss2_ref[0, 0] = f32(0)
    loss_ref[0, 0] = loss1 + f32(0)
    gtm_ref[...] = 1.0 - jnp.concatenate([mr_r[...], ms_r[...]], axis=1)


def _pairs(mask_ref_v, mask_src_v, mg, refn, srcn, gx, gy, ov):
    args = (mask_ref_v.reshape(1, 1024), mask_src_v.reshape(1, 1024),
            mg.reshape(2048, 1), mg.reshape(1, 2048),
            refn.reshape(512, 1), refn.reshape(1, 512),
            srcn.reshape(512, 1), srcn.reshape(1, 512),
            gx.reshape(2048, 1), gx.reshape(1, 2048),
            gy.reshape(2048, 1), gy.reshape(1, 2048),
            ov.reshape(2048, 1), ov.reshape(1, 2048))
    specs = [pl.BlockSpec(a.shape, lambda: (0, 0)) for a in args]
    return pl.pallas_call(
        _pairs_body,
        in_specs=specs,
        out_specs=[
            pl.BlockSpec(memory_space=pltpu.SMEM),
            pl.BlockSpec(memory_space=pltpu.SMEM),
            pl.BlockSpec(memory_space=pltpu.SMEM),
            pl.BlockSpec((1, 2048), lambda: (0, 0)),
        ],
        out_shape=[
            jax.ShapeDtypeStruct((1, 1), jnp.float32),
            jax.ShapeDtypeStruct((1, 1), jnp.float32),
            jax.ShapeDtypeStruct((1, 1), jnp.float32),
            jax.ShapeDtypeStruct((1, 2048), jnp.float32),
        ],
    )(*args)


def kernel(src_points, ref_points, src_points_c, ref_points_c,
           src_node_corr_indices, ref_node_corr_indices,
           gt_node_corr_indices, gt_node_corr_overlaps,
           src_back_indices, ref_back_indices):
    bm_src, bm_ref = _sc_scatter(src_back_indices, ref_back_indices)
    idx_src, idx_ref = _nn_argmin(src_points_c, src_points,
                                  ref_points_c, ref_points)
    gx = gt_node_corr_indices[:, 0]
    gy = gt_node_corr_indices[:, 1]
    mask_src, mask_ref, mgx, mgy = _sc_gather(
        bm_src, bm_ref, idx_src.reshape(-1), idx_ref.reshape(-1), gx, gy)
    loss, loss1, loss2, inv_gtm = _pairs(
        mask_ref.reshape(-1), mask_src.reshape(-1), mgx * mgy,
        ref_node_corr_indices, src_node_corr_indices,
        gx, gy, gt_node_corr_overlaps)
    return (loss.reshape(()), loss1.reshape(()), loss2.reshape(()),
            inv_gtm.reshape(2048))
